# baseline (speedup 1.0000x reference)
import functools

import jax
import jax.numpy as jnp
from jax import lax
from jax.experimental import pallas as pl
from jax.experimental.pallas import tpu as pltpu
from jax.experimental.pallas import tpu_sc as plsc

B = 4096
L = 200
LPAD = 208
NCHUNK = 2
CHUNK = LPAD // NCHUNK
EMBED = 64
OUT = 192


def _sc_body(table_hbm, x_hbm, out_hbm, x_v, buf0, buf1, out_v, sem0, sem1,
             *, rows_per_worker, num_cores):
    wid = lax.axis_index("s") * num_cores + lax.axis_index("c")
    base = wid * rows_per_worker

    pltpu.sync_copy(x_hbm.at[pl.ds(base, rows_per_worker)], x_v)

    bufs = (buf0, buf1)
    sems = (sem0, sem1)

    for c in range(NCHUNK):
        pltpu.make_async_copy(
            table_hbm.at[x_v.at[0, c]], bufs[c], sems[c]).start()

    def row_body(r, _):
        accs = tuple(jnp.zeros((16,), jnp.float32) for _ in range(4))
        for c in range(NCHUNK):
            buf, sem = bufs[c], sems[c]
            pltpu.make_async_copy(
                table_hbm.at[x_v.at[r, c]], buf, sem).wait()

            def chunk_body(i, accs, buf=buf):
                t = i * 4
                for u in range(4):
                    accs = tuple(
                        accs[q] + buf[t + u, pl.ds(q * 16, 16)]
                        for q in range(4))
                return accs

            accs = lax.fori_loop(0, CHUNK // 4, chunk_body, accs)

            @pl.when(r + 1 < rows_per_worker)
            def _():
                pltpu.make_async_copy(
                    table_hbm.at[x_v.at[r + 1, c]], buf, sem).start()

        for q in range(4):
            out_v[r, pl.ds(q * 16, 16)] = accs[q]
        return 0

    lax.fori_loop(0, rows_per_worker, row_body, 0)
    pltpu.sync_copy(out_v, out_hbm.at[pl.ds(base, rows_per_worker)])


def _make_sc_gather_sum():
    mesh = plsc.VectorSubcoreMesh(core_axis_name="c", subcore_axis_name="s")
    nw = mesh.num_cores * mesh.num_subcores
    rows_per_worker = B // nw
    body = functools.partial(_sc_body, rows_per_worker=rows_per_worker,
                             num_cores=mesh.num_cores)
    return pl.kernel(
        body,
        out_type=jax.ShapeDtypeStruct((B, EMBED), jnp.float32),
        mesh=mesh,
        scratch_types=[
            pltpu.VMEM((rows_per_worker, NCHUNK, CHUNK), jnp.int32),
            pltpu.VMEM((CHUNK, EMBED), jnp.float32),
            pltpu.VMEM((CHUNK, EMBED), jnp.float32),
            pltpu.VMEM((rows_per_worker, EMBED), jnp.float32),
            pltpu.SemaphoreType.DMA,
            pltpu.SemaphoreType.DMA,
        ],
        compiler_params=pltpu.CompilerParams(use_tc_tiling_on_sc=False),
    )


def _tc_finalize_body(sums_ref, x_ref, w_ref, b_ref, o_ref):
    cnt = jnp.sum((x_ref[...] != 0).astype(jnp.float32), axis=1,
                  keepdims=True)
    pooled = sums_ref[...] / jnp.maximum(cnt, 1.0)
    o_ref[...] = (
        jnp.dot(pooled, w_ref[...], preferred_element_type=jnp.float32)
        + b_ref[...])


def _tc_finalize(sums, x, W, b2d):
    blk = 512
    grid = (B // blk,)
    return pl.pallas_call(
        _tc_finalize_body,
        grid=grid,
        in_specs=[
            pl.BlockSpec((blk, EMBED), lambda i: (i, 0)),
            pl.BlockSpec((blk, L), lambda i: (i, 0)),
            pl.BlockSpec((EMBED, OUT), lambda i: (0, 0)),
            pl.BlockSpec((1, OUT), lambda i: (0, 0)),
        ],
        out_specs=pl.BlockSpec((blk, OUT), lambda i: (i, 0)),
        out_shape=jax.ShapeDtypeStruct((B, OUT), jnp.float32),
    )(sums, x, W, b2d)


def kernel(x, emb_table, W, b):
    xp = jnp.pad(x, ((0, 0), (0, LPAD - L)))
    xr = xp.reshape(B, NCHUNK, CHUNK)
    sums = _make_sc_gather_sum()(emb_table, xr)
    return _tc_finalize(sums, x, W, b.reshape(1, OUT))
